# baseline (speedup 1.0000x reference)
import functools

import jax
import jax.numpy as jnp
from jax import lax
from jax.experimental import pallas as pl
from jax.experimental.pallas import tpu as pltpu
from jax.experimental.pallas import tpu_sc as plsc

B = 16384
NCONT = 13
NCATE = 26
NTOT = NCONT + NCATE
D = 32
NC = 2
NS = 16
NW = NC * NS
BPW = B // NW
CB = 64
NCHUNK = BPW // CB

_mesh = plsc.VectorSubcoreMesh(core_axis_name="c", subcore_axis_name="s")


@functools.partial(
    pl.kernel,
    mesh=_mesh,
    compiler_params=pltpu.CompilerParams(use_tc_tiling_on_sc=False),
    out_type=jax.ShapeDtypeStruct((B * NTOT, D), jnp.float32),
    scratch_types=[
        pltpu.VMEM((CB * NTOT,), jnp.int32),
        pltpu.VMEM((CB * NCONT + 16,), jnp.float32),
        pltpu.VMEM((CB * NTOT, D), jnp.float32),
        pltpu.SemaphoreType.DMA,
    ],
)
def _emb_lookup(cv_hbm, idx_hbm, table_hbm, out_hbm, idx_v, cv_v, stage, sem):
    wid = lax.axis_index("s") * NC + lax.axis_index("c")
    base = wid * BPW

    def chunk(g, carry):
        b0 = base + g * CB
        pltpu.sync_copy(idx_hbm.at[pl.ds(b0 * NTOT, CB * NTOT)], idx_v)
        pltpu.async_copy(table_hbm.at[idx_v], stage, sem).wait()
        pltpu.sync_copy(
            cv_hbm.at[pl.ds(b0 * NCONT, CB * NCONT)],
            cv_v.at[pl.ds(0, CB * NCONT)],
        )

        def scale_b(b, c2):
            cvb = cv_v[pl.ds(b * NCONT, 16)]
            for j in range(NCONT):
                s = cvb[j]
                row = b * NTOT + j
                for d0 in range(0, D, 16):
                    stage[row, pl.ds(d0, 16)] = stage[row, pl.ds(d0, 16)] * s
            return c2

        lax.fori_loop(0, CB, scale_b, 0)
        pltpu.sync_copy(stage, out_hbm.at[pl.ds(b0 * NTOT, CB * NTOT)])
        return carry

    lax.fori_loop(0, NCHUNK, chunk, 0)


def kernel(continuous_value, universal_category_index, emb_table, cont_idx):
    ci = cont_idx.astype(jnp.int32)
    idx39 = jnp.concatenate(
        [
            jnp.broadcast_to(ci[None, :], (B, NCONT)),
            universal_category_index.astype(jnp.int32),
        ],
        axis=1,
    ).reshape(B * NTOT)
    cv = continuous_value.reshape(B * NCONT)
    out = _emb_lookup(cv, idx39, emb_table)
    return out.reshape(B, NTOT, D)
